# Initial kernel scaffold; baseline (speedup 1.0000x reference)
import jax
import jax.numpy as jnp
from jax.experimental import pallas as pl


def kernel(raw_features, nodes, edge_index, W1, W2, W3, W4, Wout):
    raise NotImplementedError("write your pallas kernel here")


# speedup vs baseline: 9.9887x; 9.9887x over previous
import functools

import jax
import jax.numpy as jnp
from jax import lax
from jax.experimental import pallas as pl
from jax.experimental.pallas import tpu as pltpu
from jax.experimental.pallas import tpu_sc as plsc

N = 50000
E = 800000
NCLS = 16

NT = 32
CHUNK = 128
C = 196
EPT = C * CHUNK
E_PAD = NT * EPT
JUNK = N
N_ACC = 50400
NR = N_ACC // 16
ZR = 150
IB = 14

_MESH = plsc.VectorSubcoreMesh(core_axis_name="c", subcore_axis_name="s")
_SC_PARAMS = pltpu.CompilerParams(use_tc_tiling_on_sc=False)


def _make_agg(dc):
  @functools.partial(
      pl.kernel,
      out_type=jax.ShapeDtypeStruct((2, N_ACC, dc), jnp.float32),
      mesh=_MESH,
      scratch_types=[
          pltpu.VMEM_SHARED((N_ACC, dc), jnp.float32),
          pltpu.VMEM((IB, CHUNK), jnp.int32),
          pltpu.VMEM((IB, CHUNK), jnp.int32),
          pltpu.VMEM((CHUNK, dc), jnp.float32),
          pltpu.VMEM((ZR, dc), jnp.float32),
          pltpu.SemaphoreType.DMA,
      ],
      compiler_params=_SC_PARAMS,
  )
  def agg(tbl, srcs, dsts, out, acc, src_v, dst_v, rowbuf, zbuf, sem):
    cid = lax.axis_index("c")
    sid = lax.axis_index("s")
    wid = cid * 16 + sid
    z16 = jnp.zeros((16,), jnp.float32)

    @pl.loop(0, ZR)
    def _(i):
      for j in range(dc // 16):
        zbuf[i, pl.ds(j * 16, 16)] = z16

    row0 = sid * NR

    @pl.loop(0, NR // ZR)
    def _(k):
      pltpu.sync_copy(zbuf, acc.at[pl.ds(row0 + k * ZR, ZR)])

    plsc.subcore_barrier()

    @pl.loop(0, C // IB)
    def _(b):
      pltpu.sync_copy(srcs.at[wid, pl.ds(b * IB, IB)], src_v)
      pltpu.sync_copy(dsts.at[wid, pl.ds(b * IB, IB)], dst_v)

      @pl.loop(0, IB)
      def _(j):
        pltpu.async_copy(tbl.at[src_v.at[j]], rowbuf, sem).wait()
        pltpu.sync_copy(rowbuf, acc.at[dst_v.at[j]], add=True)

    plsc.subcore_barrier()
    pltpu.sync_copy(acc.at[pl.ds(row0, NR)], out.at[cid, pl.ds(row0, NR)])

  return agg


_agg16 = _make_agg(16)
_agg32 = _make_agg(32)


@functools.partial(
    pl.kernel,
    out_type=jax.ShapeDtypeStruct((NT * CHUNK, NCLS), jnp.float32),
    mesh=_MESH,
    scratch_types=[
        pltpu.VMEM((CHUNK,), jnp.int32),
        pltpu.VMEM((CHUNK, NCLS), jnp.float32),
        pltpu.SemaphoreType.DMA,
    ],
    compiler_params=_SC_PARAMS,
)
def _gather_rows(scores, nodes2d, out, idx_v, rowbuf, sem):
  cid = lax.axis_index("c")
  sid = lax.axis_index("s")
  wid = cid * 16 + sid
  pltpu.sync_copy(nodes2d.at[wid], idx_v)
  pltpu.async_copy(scores.at[idx_v], rowbuf, sem).wait()
  pltpu.sync_copy(rowbuf, out.at[pl.ds(wid * CHUNK, CHUNK)])


R = 2000
_GRID = N // R


def _row_spec(d):
  return pl.BlockSpec((R, d), lambda i: (i, 0))


def _full_spec(shape):
  return pl.BlockSpec(shape, lambda i: tuple(0 for _ in shape))


def _tc1_body(t1, p0, p1, wt, wb, h2, invd):
  s = p0[...] + p1[...]
  deg = s[:, 3:4] + 1.0
  inv = 1.0 / deg
  neigh = (s + t1[...]) * inv
  acc = jnp.dot(t1[...], wt[...], preferred_element_type=jnp.float32)
  acc += jnp.dot(neigh, wb[...], preferred_element_type=jnp.float32)
  h2[...] = jnp.maximum(acc, 0.0)
  invd[...] = inv


def _tc2_body(h, p0, p1, invd, wt, wb, outa, outb):
  neigh = (p0[...] + p1[...] + h[...]) * invd[...]
  acc = jnp.dot(h[...], wt[...], preferred_element_type=jnp.float32)
  acc += jnp.dot(neigh, wb[...], preferred_element_type=jnp.float32)
  o = jnp.maximum(acc, 0.0)
  outa[...] = o[:, :32]
  outb[...] = o[:, 32:]


def _tc3_body(ha, hb, pa0, pa1, pb0, pb1, invd, w, outa, outb):
  inv = invd[...]
  wm = w[...]
  na = (pa0[...] + pa1[...] + ha[...]) * inv
  nb = (pb0[...] + pb1[...] + hb[...]) * inv
  acc = jnp.dot(ha[...], wm[0:32], preferred_element_type=jnp.float32)
  acc += jnp.dot(hb[...], wm[32:64], preferred_element_type=jnp.float32)
  acc += jnp.dot(na, wm[64:96], preferred_element_type=jnp.float32)
  acc += jnp.dot(nb, wm[96:128], preferred_element_type=jnp.float32)
  o = jnp.maximum(acc, 0.0)
  outa[...] = o[:, :32]
  outb[...] = o[:, 32:]


def _tc4_body(ha, hb, pa0, pa1, pb0, pb1, invd, w, wout, scores):
  inv = invd[...]
  wm = w[...]
  na = (pa0[...] + pa1[...] + ha[...]) * inv
  nb = (pb0[...] + pb1[...] + hb[...]) * inv
  acc = jnp.dot(ha[...], wm[0:32], preferred_element_type=jnp.float32)
  acc += jnp.dot(hb[...], wm[32:64], preferred_element_type=jnp.float32)
  acc += jnp.dot(na, wm[64:96], preferred_element_type=jnp.float32)
  acc += jnp.dot(nb, wm[96:128], preferred_element_type=jnp.float32)
  h5 = jnp.maximum(acc, 0.0)
  scores[...] = jnp.dot(h5, wout[...], preferred_element_type=jnp.float32)


def _sds(shape):
  return jax.ShapeDtypeStruct(shape, jnp.float32)


_tc1 = pl.pallas_call(
    _tc1_body,
    grid=(_GRID,),
    in_specs=[_row_spec(16), _row_spec(16), _row_spec(16),
              _full_spec((16, 32)), _full_spec((16, 32))],
    out_specs=[_row_spec(32), _row_spec(1)],
    out_shape=[_sds((N, 32)), _sds((N, 1))],
)

_tc2 = pl.pallas_call(
    _tc2_body,
    grid=(_GRID,),
    in_specs=[_row_spec(32), _row_spec(32), _row_spec(32), _row_spec(1),
              _full_spec((32, 64)), _full_spec((32, 64))],
    out_specs=[_row_spec(32), _row_spec(32)],
    out_shape=[_sds((N, 32)), _sds((N, 32))],
)

_tc3 = pl.pallas_call(
    _tc3_body,
    grid=(_GRID,),
    in_specs=[_row_spec(32)] * 6 + [_row_spec(1), _full_spec((128, 64))],
    out_specs=[_row_spec(32), _row_spec(32)],
    out_shape=[_sds((N, 32)), _sds((N, 32))],
)

_tc4 = pl.pallas_call(
    _tc4_body,
    grid=(_GRID,),
    in_specs=[_row_spec(32)] * 6 + [_row_spec(1), _full_spec((128, 32)),
                                    _full_spec((32, NCLS))],
    out_specs=_row_spec(NCLS),
    out_shape=_sds((N, NCLS)),
)


def kernel(raw_features, nodes, edge_index, W1, W2, W3, W4, Wout):
  f32 = jnp.float32
  pad = E_PAD - E
  src = jnp.concatenate(
      [edge_index[0], jnp.zeros((pad,), jnp.int32)]).reshape(NT, C, CHUNK)
  dst = jnp.concatenate(
      [edge_index[1], jnp.full((pad,), JUNK, jnp.int32)]).reshape(NT, C, CHUNK)

  t1 = jnp.concatenate(
      [raw_features, jnp.ones((N, 1), f32), jnp.zeros((N, 12), f32)], axis=1)
  w1t = jnp.zeros((16, 32), f32).at[0:3].set(W1[0:3])
  w1b = jnp.zeros((16, 32), f32).at[0:3].set(W1[3:6])

  p1 = _agg16(t1, src, dst)
  h2, invd = _tc1(t1, p1[0, :N], p1[1, :N], w1t, w1b)

  p2 = _agg32(h2, src, dst)
  h3a, h3b = _tc2(h2, p2[0, :N], p2[1, :N], invd, W2[:32], W2[32:])

  p3a = _agg32(h3a, src, dst)
  p3b = _agg32(h3b, src, dst)
  h4a, h4b = _tc3(h3a, h3b, p3a[0, :N], p3a[1, :N], p3b[0, :N], p3b[1, :N],
                  invd, W3)

  p4a = _agg32(h4a, src, dst)
  p4b = _agg32(h4b, src, dst)
  scores_all = _tc4(h4a, h4b, p4a[0, :N], p4a[1, :N], p4b[0, :N], p4b[1, :N],
                    invd, W4, Wout)

  return _gather_rows(scores_all, nodes.reshape(NT, CHUNK))
